# Initial kernel scaffold; baseline (speedup 1.0000x reference)
import jax
import jax.numpy as jnp
from jax.experimental import pallas as pl


def kernel(feat, member_idx, cluster_mask, pe_idx, global_attn, Wq, bq, Wkv, bkv, blank_k, blank_v, Wpe, bpe, Wproj, bproj, pre_table):
    raise NotImplementedError("write your pallas kernel here")


# speedup vs baseline: 27.7895x; 27.7895x over previous
import functools

import jax
import jax.numpy as jnp
from jax import lax
from jax.experimental import pallas as pl
from jax.experimental.pallas import tpu as pltpu
from jax.experimental.pallas import tpu_sc as plsc

N = 2048
C = 768
H = 12
CH = C // H
T = 2401
TP = 2432
NP = H // 2
NN = N * N
SCALE = CH ** -0.5
QT = 128
NQ = N // QT
NWORK = 32
CHUNK = NN // NWORK
IB = 4096


def _proj_body(feat_ref, wq_ref, bq_ref, wkv_ref, bkv_ref, pre_ref, wpe_ref,
               bpe_ref, q_ref, kv_ref, pe_ref):
    f = feat_ref[...]
    q_ref[...] = jnp.dot(f, wq_ref[...],
                         preferred_element_type=jnp.float32) + bq_ref[...]
    kv_ref[...] = jnp.dot(f, wkv_ref[...],
                          preferred_element_type=jnp.float32) + bkv_ref[...]

    @pl.when(pl.program_id(0) == 0)
    def _():
        pe_ref[...] = jnp.dot(pre_ref[...], wpe_ref[...],
                              preferred_element_type=jnp.float32) + bpe_ref[...]


_proj_call = pl.pallas_call(
    _proj_body,
    grid=(NQ,),
    in_specs=[
        pl.BlockSpec((QT, C), lambda i: (i, 0)),
        pl.BlockSpec((C, C), lambda i: (0, 0)),
        pl.BlockSpec((1, C), lambda i: (0, 0)),
        pl.BlockSpec((C, 2 * C), lambda i: (0, 0)),
        pl.BlockSpec((1, 2 * C), lambda i: (0, 0)),
        pl.BlockSpec((TP, 8), lambda i: (0, 0)),
        pl.BlockSpec((8, 16), lambda i: (0, 0)),
        pl.BlockSpec((1, 16), lambda i: (0, 0)),
    ],
    out_specs=[
        pl.BlockSpec((QT, C), lambda i: (i, 0)),
        pl.BlockSpec((QT, 2 * C), lambda i: (i, 0)),
        pl.BlockSpec((TP, 16), lambda i: (0, 0)),
    ],
    out_shape=[
        jax.ShapeDtypeStruct((N, C), jnp.float32),
        jax.ShapeDtypeStruct((N, 2 * C), jnp.float32),
        jax.ShapeDtypeStruct((TP, 16), jnp.float32),
    ],
)


def _sc_gather_body(idx_hbm, tab_hbm, out_hbm, tab_v, idx_v, out_v):
    wid = lax.axis_index("s") * 2 + lax.axis_index("c")
    base = wid * CHUNK
    pltpu.sync_copy(tab_hbm, tab_v)

    def chunk_body(c, carry):
        off = base + c * IB
        pltpu.sync_copy(idx_hbm.at[pl.ds(off, IB)], idx_v)

        def vec_body(i, carry2):
            ids = idx_v[pl.ds(i * 16, 16)]
            for j in range(NP):
                g = plsc.load_gather(tab_v, [ids + (j * TP)])
                out_v[j, pl.ds(i * 16, 16)] = g
            return carry2

        lax.fori_loop(0, IB // 16, vec_body, 0, unroll=4)
        pltpu.sync_copy(out_v, out_hbm.at[:, pl.ds(off, IB)])
        return carry

    lax.fori_loop(0, CHUNK // IB, chunk_body, 0)


@functools.lru_cache(maxsize=1)
def _gather_call():
    return pl.kernel(
        _sc_gather_body,
        out_type=jax.ShapeDtypeStruct((NP, NN), jnp.int32),
        mesh=plsc.VectorSubcoreMesh(core_axis_name="c", subcore_axis_name="s"),
        scratch_types=[
            pltpu.VMEM((NP * TP,), jnp.int32),
            pltpu.VMEM((IB,), jnp.int32),
            pltpu.VMEM((NP, IB), jnp.int32),
        ],
        compiler_params=pltpu.CompilerParams(needs_layout_passes=False),
    )


def _attn_body(q_ref, kv_ref, bias_ref, bk_ref, bv_ref, wp_ref, bp_ref,
               out_ref):
    outs = []
    for h in range(H):
        qh = q_ref[:, h * CH:(h + 1) * CH]
        kh = kv_ref[:, h * 2 * CH:h * 2 * CH + CH]
        vh = kv_ref[:, h * 2 * CH + CH:(h + 1) * 2 * CH]
        s = lax.dot_general(qh, kh, (((1,), (1,)), ((), ())),
                            preferred_element_type=jnp.float32) * SCALE
        u = bias_ref[h // 2]
        if h % 2 == 0:
            fb = lax.shift_left(u, 16)
        else:
            fb = lax.bitwise_and(u, jnp.int32(-65536))
        s = s + lax.bitcast_convert_type(fb, jnp.float32)
        bkh = bk_ref[:, h * CH:(h + 1) * CH]
        sbl = lax.dot_general(qh, bkh, (((1,), (1,)), ((), ())),
                              preferred_element_type=jnp.float32) * SCALE
        m = jnp.maximum(jnp.max(s, axis=1, keepdims=True), sbl)
        p = jnp.exp(s - m)
        pb = jnp.exp(sbl - m)
        den = jnp.sum(p, axis=1, keepdims=True) + pb
        o = lax.dot_general(p, vh, (((1,), (0,)), ((), ())),
                            preferred_element_type=jnp.float32)
        o = (o + pb * bv_ref[:, h * CH:(h + 1) * CH]) / den
        outs.append(o)
    oc = jnp.concatenate(outs, axis=1)
    out_ref[...] = lax.dot_general(oc, wp_ref[...], (((1,), (0,)), ((), ())),
                                   preferred_element_type=jnp.float32) + bp_ref[...]


_attn_call = pl.pallas_call(
    _attn_body,
    grid=(NQ,),
    in_specs=[
        pl.BlockSpec((QT, C), lambda i: (i, 0)),
        pl.BlockSpec((N, 2 * C), lambda i: (0, 0)),
        pl.BlockSpec((NP, QT, N), lambda i: (0, i, 0)),
        pl.BlockSpec((1, C), lambda i: (0, 0)),
        pl.BlockSpec((1, C), lambda i: (0, 0)),
        pl.BlockSpec((C, C), lambda i: (0, 0)),
        pl.BlockSpec((1, C), lambda i: (0, 0)),
    ],
    out_specs=pl.BlockSpec((QT, C), lambda i: (i, 0)),
    out_shape=jax.ShapeDtypeStruct((N, C), jnp.float32),
)


def kernel(feat, member_idx, cluster_mask, pe_idx, global_attn, Wq, bq, Wkv,
           bkv, blank_k, blank_v, Wpe, bpe, Wproj, bproj, pre_table):
    del member_idx, cluster_mask, global_attn
    feat2 = feat.reshape(N, C)
    pre_pad = jnp.pad(pre_table, ((0, TP - T), (0, 3)))
    wpe_pad = jnp.pad(Wpe, ((0, 3), (0, 4)))
    bpe_pad = jnp.pad(bpe, (0, 4)).reshape(1, 16)
    q, kv, pe = _proj_call(feat2, Wq, bq.reshape(1, C), Wkv,
                           bkv.reshape(1, 2 * C), pre_pad, wpe_pad, bpe_pad)
    pe_bf = pe[:, :H].astype(jnp.bfloat16)
    u16 = lax.bitcast_convert_type(pe_bf, jnp.uint16).astype(jnp.uint32)
    tab = (u16[:, 0::2] | (u16[:, 1::2] << 16)).astype(jnp.int32).T
    bias_pack = _gather_call()(pe_idx.reshape(NN), tab.reshape(NP * TP))
    out = _attn_call(q, kv, bias_pack.reshape(NP, N, N),
                     blank_k.reshape(1, C), blank_v.reshape(1, C), Wproj,
                     bproj.reshape(1, C))
    return out.reshape(1, N, C)


# speedup vs baseline: 44.0577x; 1.5854x over previous
import functools

import jax
import jax.numpy as jnp
from jax import lax
from jax.experimental import pallas as pl
from jax.experimental.pallas import tpu as pltpu
from jax.experimental.pallas import tpu_sc as plsc

N = 2048
C = 768
H = 12
CH = C // H
T = 2401
TP = 2432
NP = 3
NN = N * N
SCALE = CH ** -0.5
QT = 128
NQ = N // QT
NWORK = 32
ROWS_W = N // NWORK
SR = 4
NHS = ROWS_W // SR


def _proj_body(feat_ref, wq_ref, bq_ref, wkv_ref, bkv_ref, q_ref, kv_ref):
    f = feat_ref[...]
    q_ref[...] = jnp.dot(f, wq_ref[...],
                         preferred_element_type=jnp.float32) + bq_ref[...]
    kv_ref[...] = jnp.dot(f, wkv_ref[...],
                          preferred_element_type=jnp.float32) + bkv_ref[...]


_proj_call = pl.pallas_call(
    _proj_body,
    grid=(NQ,),
    in_specs=[
        pl.BlockSpec((QT, C), lambda i: (i, 0)),
        pl.BlockSpec((C, C), lambda i: (0, 0)),
        pl.BlockSpec((1, C), lambda i: (0, 0)),
        pl.BlockSpec((C, 2 * C), lambda i: (0, 0)),
        pl.BlockSpec((1, 2 * C), lambda i: (0, 0)),
    ],
    out_specs=[
        pl.BlockSpec((QT, C), lambda i: (i, 0)),
        pl.BlockSpec((QT, 2 * C), lambda i: (i, 0)),
    ],
    out_shape=[
        jax.ShapeDtypeStruct((N, C), jnp.float32),
        jax.ShapeDtypeStruct((N, 2 * C), jnp.float32),
    ],
)


def _sc_gather_body(idx_hbm, tab_hbm, out_hbm, tab_v, idx_v, out_v, in_sem,
                    out_sem):
    wid = lax.axis_index("s") * 2 + lax.axis_index("c")
    row0 = wid * ROWS_W
    pltpu.sync_copy(tab_hbm, tab_v)

    def in_copy(hs, b):
        return pltpu.make_async_copy(
            idx_hbm.at[pl.ds(row0 + hs * SR, SR), :], idx_v.at[b],
            in_sem.at[b])

    def out_copy(hs, b):
        return pltpu.make_async_copy(
            out_v.at[b], out_hbm.at[:, pl.ds(row0 + hs * SR, SR), :],
            out_sem.at[b])

    in_copy(0, 0).start()
    for hs in range(NHS):
        b = hs % 2
        if hs + 1 < NHS:
            in_copy(hs + 1, 1 - b).start()
        in_copy(hs, b).wait()
        if hs >= 2:
            out_copy(hs - 2, b).wait()
        for r in range(SR):

            def vec_body(i, carry, _b=b, _r=r):
                ids = idx_v[_b, _r, pl.ds(i * 16, 16)]
                for j in range(NP):
                    g = plsc.load_gather(tab_v, [ids + (j * TP)])
                    out_v[_b, j, _r, pl.ds(i * 16, 16)] = g
                return carry

            lax.fori_loop(0, N // 16, vec_body, 0, unroll=4)
        out_copy(hs, b).start()
    out_copy(NHS - 2, 0).wait()
    out_copy(NHS - 1, 1).wait()


@functools.lru_cache(maxsize=1)
def _gather_call():
    return pl.kernel(
        _sc_gather_body,
        out_type=jax.ShapeDtypeStruct((NP, N, N), jnp.int32),
        mesh=plsc.VectorSubcoreMesh(core_axis_name="c", subcore_axis_name="s"),
        scratch_types=[
            pltpu.VMEM((NP * TP,), jnp.int32),
            pltpu.VMEM((2, SR, N), jnp.int32),
            pltpu.VMEM((2, NP, SR, N), jnp.int32),
            pltpu.SemaphoreType.DMA((2,)),
            pltpu.SemaphoreType.DMA((2,)),
        ],
        compiler_params=pltpu.CompilerParams(needs_layout_passes=False,
                                             use_tc_tiling_on_sc=True),
    )


def _attn_body(q_ref, kv_ref, bias_ref, wpe_ref, bpe_ref, bk_ref, bv_ref,
               wp_ref, bp_ref, out_ref):
    u0 = bias_ref[0]
    u1 = bias_ref[1]
    u2 = bias_ref[2]
    planes = (
        lax.bitcast_convert_type(lax.shift_left(u0, 16), jnp.float32),
        lax.bitcast_convert_type(lax.bitwise_and(u0, jnp.int32(-65536)),
                                 jnp.float32),
        lax.bitcast_convert_type(lax.shift_left(u1, 16), jnp.float32),
        lax.bitcast_convert_type(lax.bitwise_and(u1, jnp.int32(-65536)),
                                 jnp.float32),
        lax.bitcast_convert_type(lax.shift_left(u2, 16), jnp.float32),
    )
    outs = []
    for h in range(H):
        qh = q_ref[:, h * CH:(h + 1) * CH]
        kh = kv_ref[:, h * 2 * CH:h * 2 * CH + CH]
        vh = kv_ref[:, h * 2 * CH + CH:(h + 1) * 2 * CH]
        s = lax.dot_general(qh, kh, (((1,), (1,)), ((), ())),
                            preferred_element_type=jnp.float32) * SCALE
        b_h = planes[0] * wpe_ref[0, h] + planes[1] * wpe_ref[1, h]
        b_h += planes[2] * wpe_ref[2, h] + planes[3] * wpe_ref[3, h]
        b_h += planes[4] * wpe_ref[4, h] + bpe_ref[0, h]
        s = s + b_h
        bkh = bk_ref[:, h * CH:(h + 1) * CH]
        sbl = lax.dot_general(qh, bkh, (((1,), (1,)), ((), ())),
                              preferred_element_type=jnp.float32) * SCALE
        m = jnp.maximum(jnp.max(s, axis=1, keepdims=True), sbl)
        p = jnp.exp(s - m)
        pb = jnp.exp(sbl - m)
        den = jnp.sum(p, axis=1, keepdims=True) + pb
        o = lax.dot_general(p, vh, (((1,), (0,)), ((), ())),
                            preferred_element_type=jnp.float32)
        o = (o + pb * bv_ref[:, h * CH:(h + 1) * CH]) / den
        outs.append(o)
    oc = jnp.concatenate(outs, axis=1)
    out_ref[...] = lax.dot_general(oc, wp_ref[...], (((1,), (0,)), ((), ())),
                                   preferred_element_type=jnp.float32) + bp_ref[...]


_attn_call = pl.pallas_call(
    _attn_body,
    grid=(NQ,),
    in_specs=[
        pl.BlockSpec((QT, C), lambda i: (i, 0)),
        pl.BlockSpec((N, 2 * C), lambda i: (0, 0)),
        pl.BlockSpec((NP, QT, N), lambda i: (0, i, 0)),
        pl.BlockSpec((5, H), lambda i: (0, 0)),
        pl.BlockSpec((1, H), lambda i: (0, 0)),
        pl.BlockSpec((1, C), lambda i: (0, 0)),
        pl.BlockSpec((1, C), lambda i: (0, 0)),
        pl.BlockSpec((C, C), lambda i: (0, 0)),
        pl.BlockSpec((1, C), lambda i: (0, 0)),
    ],
    out_specs=pl.BlockSpec((QT, C), lambda i: (i, 0)),
    out_shape=jax.ShapeDtypeStruct((N, C), jnp.float32),
)


def kernel(feat, member_idx, cluster_mask, pe_idx, global_attn, Wq, bq, Wkv,
           bkv, blank_k, blank_v, Wpe, bpe, Wproj, bproj, pre_table):
    del member_idx, cluster_mask, global_attn
    feat2 = feat.reshape(N, C)
    q, kv = _proj_call(feat2, Wq, bq.reshape(1, C), Wkv,
                       bkv.reshape(1, 2 * C))
    pre_bf = jnp.pad(pre_table, ((0, TP - T), (0, 1))).astype(jnp.bfloat16)
    u16 = lax.bitcast_convert_type(pre_bf, jnp.uint16).astype(jnp.uint32)
    tab = (u16[:, 0::2] | (u16[:, 1::2] << 16)).astype(jnp.int32).T
    bias_pack = _gather_call()(pe_idx.reshape(N, N), tab.reshape(NP * TP))
    out = _attn_call(q, kv, bias_pack, Wpe, bpe.reshape(1, H),
                     blank_k.reshape(1, C), blank_v.reshape(1, C), Wproj,
                     bproj.reshape(1, C))
    return out.reshape(1, N, C)


# speedup vs baseline: 53.1437x; 1.2062x over previous
import functools

import jax
import jax.numpy as jnp
from jax import lax
from jax.experimental import pallas as pl
from jax.experimental.pallas import tpu as pltpu
from jax.experimental.pallas import tpu_sc as plsc

N = 2048
C = 768
H = 12
CH = C // H
T = 2401
TP = 2432
NP = 3
NN = N * N
SCALE = CH ** -0.5
QT = 128
NQ = N // QT
NWORK = 32
NSPLIT = 2
RSP = N // NSPLIT
ROWS_W = RSP // NWORK
SR = 4
NHS = ROWS_W // SR


def _proj_body(feat_ref, wq_ref, bq_ref, wkv_ref, bkv_ref, q_ref, kv_ref):
    f = feat_ref[...]
    q_ref[...] = jnp.dot(f, wq_ref[...],
                         preferred_element_type=jnp.float32) + bq_ref[...]
    kv_ref[...] = jnp.dot(f, wkv_ref[...],
                          preferred_element_type=jnp.float32) + bkv_ref[...]


_proj_call = pl.pallas_call(
    _proj_body,
    grid=(NQ,),
    in_specs=[
        pl.BlockSpec((QT, C), lambda i: (i, 0)),
        pl.BlockSpec((C, C), lambda i: (0, 0)),
        pl.BlockSpec((1, C), lambda i: (0, 0)),
        pl.BlockSpec((C, 2 * C), lambda i: (0, 0)),
        pl.BlockSpec((1, 2 * C), lambda i: (0, 0)),
    ],
    out_specs=[
        pl.BlockSpec((QT, C), lambda i: (i, 0)),
        pl.BlockSpec((QT, 2 * C), lambda i: (i, 0)),
    ],
    out_shape=[
        jax.ShapeDtypeStruct((N, C), jnp.float32),
        jax.ShapeDtypeStruct((N, 2 * C), jnp.float32),
    ],
)


def _sc_gather_body(base_row, idx_hbm, tab_hbm, out_hbm, tab_v, idx_v, out_v,
                    in_sem, out_sem):
    wid = lax.axis_index("s") * 2 + lax.axis_index("c")
    row0 = wid * ROWS_W
    pltpu.sync_copy(tab_hbm, tab_v)

    def in_copy(hs, b):
        return pltpu.make_async_copy(
            idx_hbm.at[pl.ds(base_row + row0 + hs * SR, SR), :], idx_v.at[b],
            in_sem.at[b])

    def out_copy(hs, b):
        return pltpu.make_async_copy(
            out_v.at[b], out_hbm.at[:, pl.ds(row0 + hs * SR, SR), :],
            out_sem.at[b])

    in_copy(0, 0).start()
    for hs in range(NHS):
        b = hs % 2
        if hs + 1 < NHS:
            in_copy(hs + 1, 1 - b).start()
        in_copy(hs, b).wait()
        if hs >= 2:
            out_copy(hs - 2, b).wait()
        for r in range(SR):

            def vec_body(i, carry, _b=b, _r=r):
                ids = idx_v[_b, _r, pl.ds(i * 16, 16)]
                for j in range(NP):
                    g = plsc.load_gather(tab_v, [ids + (j * TP)])
                    out_v[_b, j, _r, pl.ds(i * 16, 16)] = g
                return carry

            lax.fori_loop(0, N // 16, vec_body, 0, unroll=4)
        out_copy(hs, b).start()
    out_copy(NHS - 2, 0).wait()
    out_copy(NHS - 1, 1).wait()


@functools.lru_cache(maxsize=None)
def _gather_call(split):
    return pl.kernel(
        functools.partial(_sc_gather_body, split * RSP),
        out_type=jax.ShapeDtypeStruct((NP, RSP, N), jnp.int32),
        mesh=plsc.VectorSubcoreMesh(core_axis_name="c", subcore_axis_name="s"),
        scratch_types=[
            pltpu.VMEM((NP * TP,), jnp.int32),
            pltpu.VMEM((2, SR, N), jnp.int32),
            pltpu.VMEM((2, NP, SR, N), jnp.int32),
            pltpu.SemaphoreType.DMA((2,)),
            pltpu.SemaphoreType.DMA((2,)),
        ],
        compiler_params=pltpu.CompilerParams(needs_layout_passes=False,
                                             use_tc_tiling_on_sc=True),
    )


def _attn_body(q_ref, kv_ref, bias_ref, wpe_ref, bpe_ref, bk_ref, bv_ref,
               wp_ref, bp_ref, out_ref):
    u0 = bias_ref[0]
    u1 = bias_ref[1]
    u2 = bias_ref[2]
    planes = (
        lax.bitcast_convert_type(lax.shift_left(u0, 16), jnp.float32),
        lax.bitcast_convert_type(lax.bitwise_and(u0, jnp.int32(-65536)),
                                 jnp.float32),
        lax.bitcast_convert_type(lax.shift_left(u1, 16), jnp.float32),
        lax.bitcast_convert_type(lax.bitwise_and(u1, jnp.int32(-65536)),
                                 jnp.float32),
        lax.bitcast_convert_type(lax.shift_left(u2, 16), jnp.float32),
    )
    outs = []
    for h in range(H):
        qh = q_ref[:, h * CH:(h + 1) * CH]
        kh = kv_ref[:, h * 2 * CH:h * 2 * CH + CH]
        vh = kv_ref[:, h * 2 * CH + CH:(h + 1) * 2 * CH]
        s = lax.dot_general(qh, kh, (((1,), (1,)), ((), ())),
                            preferred_element_type=jnp.float32) * SCALE
        b_h = planes[0] * wpe_ref[0, h] + planes[1] * wpe_ref[1, h]
        b_h += planes[2] * wpe_ref[2, h] + planes[3] * wpe_ref[3, h]
        b_h += planes[4] * wpe_ref[4, h] + bpe_ref[0, h]
        s = s + b_h
        bkh = bk_ref[:, h * CH:(h + 1) * CH]
        sbl = lax.dot_general(qh, bkh, (((1,), (1,)), ((), ())),
                              preferred_element_type=jnp.float32) * SCALE
        m = jnp.maximum(jnp.max(s, axis=1, keepdims=True), sbl)
        p = jnp.exp(s - m)
        pb = jnp.exp(sbl - m)
        den = jnp.sum(p, axis=1, keepdims=True) + pb
        o = lax.dot_general(p, vh, (((1,), (0,)), ((), ())),
                            preferred_element_type=jnp.float32)
        o = (o + pb * bv_ref[:, h * CH:(h + 1) * CH]) / den
        outs.append(o)
    oc = jnp.concatenate(outs, axis=1)
    out_ref[...] = lax.dot_general(oc, wp_ref[...], (((1,), (0,)), ((), ())),
                                   preferred_element_type=jnp.float32) + bp_ref[...]


@functools.lru_cache(maxsize=None)
def _attn_call(split):
    off = split * (RSP // QT)
    return pl.pallas_call(
        _attn_body,
        grid=(RSP // QT,),
        in_specs=[
            pl.BlockSpec((QT, C), lambda i, _o=off: (i + _o, 0)),
            pl.BlockSpec((N, 2 * C), lambda i: (0, 0)),
            pl.BlockSpec((NP, QT, N), lambda i: (0, i, 0)),
            pl.BlockSpec((5, H), lambda i: (0, 0)),
            pl.BlockSpec((1, H), lambda i: (0, 0)),
            pl.BlockSpec((1, C), lambda i: (0, 0)),
            pl.BlockSpec((1, C), lambda i: (0, 0)),
            pl.BlockSpec((C, C), lambda i: (0, 0)),
            pl.BlockSpec((1, C), lambda i: (0, 0)),
        ],
        out_specs=pl.BlockSpec((QT, C), lambda i: (i, 0)),
        out_shape=jax.ShapeDtypeStruct((RSP, C), jnp.float32),
    )


def kernel(feat, member_idx, cluster_mask, pe_idx, global_attn, Wq, bq, Wkv,
           bkv, blank_k, blank_v, Wpe, bpe, Wproj, bproj, pre_table):
    del member_idx, cluster_mask, global_attn
    feat2 = feat.reshape(N, C)
    q, kv = _proj_call(feat2, Wq, bq.reshape(1, C), Wkv,
                       bkv.reshape(1, 2 * C))
    pre_bf = jnp.pad(pre_table, ((0, TP - T), (0, 1))).astype(jnp.bfloat16)
    u16 = lax.bitcast_convert_type(pre_bf, jnp.uint16).astype(jnp.uint32)
    tab = (u16[:, 0::2] | (u16[:, 1::2] << 16)).astype(jnp.int32).T
    idx2 = pe_idx.reshape(N, N)
    tab_flat = tab.reshape(NP * TP)
    packs = [_gather_call(s)(idx2, tab_flat) for s in range(NSPLIT)]
    outs = [
        _attn_call(s)(q, kv, packs[s], Wpe, bpe.reshape(1, H),
                      blank_k.reshape(1, C), blank_v.reshape(1, C), Wproj,
                      bproj.reshape(1, C)) for s in range(NSPLIT)
    ]
    out = jnp.concatenate(outs, axis=0)
    return out.reshape(1, N, C)


# speedup vs baseline: 57.2339x; 1.0770x over previous
import functools

import jax
import jax.numpy as jnp
from jax import lax
from jax.experimental import pallas as pl
from jax.experimental.pallas import tpu as pltpu
from jax.experimental.pallas import tpu_sc as plsc

N = 2048
C = 768
H = 12
CH = C // H
T = 2401
TP = 2432
NP = 3
NN = N * N
SCALE = CH ** -0.5
QT = 128
NQ = N // QT
NWORK = 32
NSPLIT = 4
RSP = N // NSPLIT
ROWS_W = RSP // NWORK
SR = 4
NHS = ROWS_W // SR


def _proj_body(feat_ref, wq_ref, bq_ref, wkv_ref, bkv_ref, q_ref, kv_ref):
    f = feat_ref[...]
    q_ref[...] = jnp.dot(f, wq_ref[...],
                         preferred_element_type=jnp.float32) + bq_ref[...]
    kv_ref[...] = jnp.dot(f, wkv_ref[...],
                          preferred_element_type=jnp.float32) + bkv_ref[...]


_proj_call = pl.pallas_call(
    _proj_body,
    grid=(NQ,),
    in_specs=[
        pl.BlockSpec((QT, C), lambda i: (i, 0)),
        pl.BlockSpec((C, C), lambda i: (0, 0)),
        pl.BlockSpec((1, C), lambda i: (0, 0)),
        pl.BlockSpec((C, 2 * C), lambda i: (0, 0)),
        pl.BlockSpec((1, 2 * C), lambda i: (0, 0)),
    ],
    out_specs=[
        pl.BlockSpec((QT, C), lambda i: (i, 0)),
        pl.BlockSpec((QT, 2 * C), lambda i: (i, 0)),
    ],
    out_shape=[
        jax.ShapeDtypeStruct((N, C), jnp.float32),
        jax.ShapeDtypeStruct((N, 2 * C), jnp.float32),
    ],
)


def _sc_gather_body(base_row, idx_hbm, tab_hbm, out_hbm, tab_v, idx_v, out_v,
                    in_sem, out_sem):
    wid = lax.axis_index("s") * 2 + lax.axis_index("c")
    row0 = wid * ROWS_W
    pltpu.sync_copy(tab_hbm, tab_v)

    def in_copy(hs, b):
        return pltpu.make_async_copy(
            idx_hbm.at[pl.ds(base_row + row0 + hs * SR, SR), :], idx_v.at[b],
            in_sem.at[b])

    def out_copy(hs, b):
        return pltpu.make_async_copy(
            out_v.at[b], out_hbm.at[:, pl.ds(row0 + hs * SR, SR), :],
            out_sem.at[b])

    in_copy(0, 0).start()
    for hs in range(NHS):
        b = hs % 2
        if hs + 1 < NHS:
            in_copy(hs + 1, 1 - b).start()
        in_copy(hs, b).wait()
        if hs >= 2:
            out_copy(hs - 2, b).wait()
        for r in range(SR):

            def vec_body(i, carry, _b=b, _r=r):
                ids = idx_v[_b, _r, pl.ds(i * 16, 16)]
                for j in range(NP):
                    g = plsc.load_gather(tab_v, [ids + (j * TP)])
                    out_v[_b, j, _r, pl.ds(i * 16, 16)] = g
                return carry

            lax.fori_loop(0, N // 16, vec_body, 0, unroll=4)
        out_copy(hs, b).start()
    out_copy(NHS - 2, 0).wait()
    out_copy(NHS - 1, 1).wait()


@functools.lru_cache(maxsize=None)
def _gather_call(split):
    return pl.kernel(
        functools.partial(_sc_gather_body, split * RSP),
        out_type=jax.ShapeDtypeStruct((NP, RSP, N), jnp.int32),
        mesh=plsc.VectorSubcoreMesh(core_axis_name="c", subcore_axis_name="s"),
        scratch_types=[
            pltpu.VMEM((NP * TP,), jnp.int32),
            pltpu.VMEM((2, SR, N), jnp.int32),
            pltpu.VMEM((2, NP, SR, N), jnp.int32),
            pltpu.SemaphoreType.DMA((2,)),
            pltpu.SemaphoreType.DMA((2,)),
        ],
        compiler_params=pltpu.CompilerParams(needs_layout_passes=False,
                                             use_tc_tiling_on_sc=True),
    )


def _attn_body(q_ref, kv_ref, bias_ref, wpe_ref, bpe_ref, bk_ref, bv_ref,
               wp_ref, bp_ref, out_ref):
    u0 = bias_ref[0]
    u1 = bias_ref[1]
    u2 = bias_ref[2]
    planes = (
        lax.bitcast_convert_type(lax.shift_left(u0, 16), jnp.float32),
        lax.bitcast_convert_type(lax.bitwise_and(u0, jnp.int32(-65536)),
                                 jnp.float32),
        lax.bitcast_convert_type(lax.shift_left(u1, 16), jnp.float32),
        lax.bitcast_convert_type(lax.bitwise_and(u1, jnp.int32(-65536)),
                                 jnp.float32),
        lax.bitcast_convert_type(lax.shift_left(u2, 16), jnp.float32),
    )
    outs = []
    for h in range(H):
        qh = q_ref[:, h * CH:(h + 1) * CH]
        kh = kv_ref[:, h * 2 * CH:h * 2 * CH + CH]
        vh = kv_ref[:, h * 2 * CH + CH:(h + 1) * 2 * CH]
        s = lax.dot_general(qh, kh, (((1,), (1,)), ((), ())),
                            preferred_element_type=jnp.float32) * SCALE
        b_h = planes[0] * wpe_ref[0, h] + planes[1] * wpe_ref[1, h]
        b_h += planes[2] * wpe_ref[2, h] + planes[3] * wpe_ref[3, h]
        b_h += planes[4] * wpe_ref[4, h] + bpe_ref[0, h]
        s = s + b_h
        bkh = bk_ref[:, h * CH:(h + 1) * CH]
        sbl = lax.dot_general(qh, bkh, (((1,), (1,)), ((), ())),
                              preferred_element_type=jnp.float32) * SCALE
        m = jnp.maximum(jnp.max(s, axis=1, keepdims=True), sbl)
        p = jnp.exp(s - m)
        pb = jnp.exp(sbl - m)
        den = jnp.sum(p, axis=1, keepdims=True) + pb
        o = lax.dot_general(p, vh, (((1,), (0,)), ((), ())),
                            preferred_element_type=jnp.float32)
        o = (o + pb * bv_ref[:, h * CH:(h + 1) * CH]) / den
        outs.append(o)
    oc = jnp.concatenate(outs, axis=1)
    out_ref[...] = lax.dot_general(oc, wp_ref[...], (((1,), (0,)), ((), ())),
                                   preferred_element_type=jnp.float32) + bp_ref[...]


@functools.lru_cache(maxsize=None)
def _attn_call(split):
    off = split * (RSP // QT)
    return pl.pallas_call(
        _attn_body,
        grid=(RSP // QT,),
        in_specs=[
            pl.BlockSpec((QT, C), lambda i, _o=off: (i + _o, 0)),
            pl.BlockSpec((N, 2 * C), lambda i: (0, 0)),
            pl.BlockSpec((NP, QT, N), lambda i: (0, i, 0)),
            pl.BlockSpec((5, H), lambda i: (0, 0)),
            pl.BlockSpec((1, H), lambda i: (0, 0)),
            pl.BlockSpec((1, C), lambda i: (0, 0)),
            pl.BlockSpec((1, C), lambda i: (0, 0)),
            pl.BlockSpec((C, C), lambda i: (0, 0)),
            pl.BlockSpec((1, C), lambda i: (0, 0)),
        ],
        out_specs=pl.BlockSpec((QT, C), lambda i: (i, 0)),
        out_shape=jax.ShapeDtypeStruct((RSP, C), jnp.float32),
    )


def kernel(feat, member_idx, cluster_mask, pe_idx, global_attn, Wq, bq, Wkv,
           bkv, blank_k, blank_v, Wpe, bpe, Wproj, bproj, pre_table):
    del member_idx, cluster_mask, global_attn
    feat2 = feat.reshape(N, C)
    q, kv = _proj_call(feat2, Wq, bq.reshape(1, C), Wkv,
                       bkv.reshape(1, 2 * C))
    pre_bf = jnp.pad(pre_table, ((0, TP - T), (0, 1))).astype(jnp.bfloat16)
    u16 = lax.bitcast_convert_type(pre_bf, jnp.uint16).astype(jnp.uint32)
    tab = (u16[:, 0::2] | (u16[:, 1::2] << 16)).astype(jnp.int32).T
    idx2 = pe_idx.reshape(N, N)
    tab_flat = tab.reshape(NP * TP)
    packs = [_gather_call(s)(idx2, tab_flat) for s in range(NSPLIT)]
    outs = [
        _attn_call(s)(q, kv, packs[s], Wpe, bpe.reshape(1, H),
                      blank_k.reshape(1, C), blank_v.reshape(1, C), Wproj,
                      bproj.reshape(1, C)) for s in range(NSPLIT)
    ]
    out = jnp.concatenate(outs, axis=0)
    return out.reshape(1, N, C)


# speedup vs baseline: 63.4704x; 1.1090x over previous
import functools

import jax
import jax.numpy as jnp
from jax import lax
from jax.experimental import pallas as pl
from jax.experimental.pallas import tpu as pltpu
from jax.experimental.pallas import tpu_sc as plsc

N = 2048
C = 768
H = 12
CH = C // H
T = 2401
TP = 2432
NP = 3
NN = N * N
SCALE = CH ** -0.5
QT = 128
NQ = N // QT
NWORK = 32
NSPLIT = 4
RSP = N // NSPLIT
ROWS_W = RSP // NWORK
SR = 4
NHS = ROWS_W // SR


def _proj_body(feat_ref, wq_ref, bq_ref, wkv_ref, bkv_ref, q_ref, kv_ref):
    f = feat_ref[...]
    q = jnp.dot(f, wq_ref[...],
                preferred_element_type=jnp.float32) + bq_ref[...]
    kv = jnp.dot(f, wkv_ref[...],
                 preferred_element_type=jnp.float32) + bkv_ref[...]
    q_ref[...] = q.astype(jnp.bfloat16)
    kv_ref[...] = kv.astype(jnp.bfloat16)


_proj_call = pl.pallas_call(
    _proj_body,
    grid=(NQ,),
    in_specs=[
        pl.BlockSpec((QT, C), lambda i: (i, 0)),
        pl.BlockSpec((C, C), lambda i: (0, 0)),
        pl.BlockSpec((1, C), lambda i: (0, 0)),
        pl.BlockSpec((C, 2 * C), lambda i: (0, 0)),
        pl.BlockSpec((1, 2 * C), lambda i: (0, 0)),
    ],
    out_specs=[
        pl.BlockSpec((QT, C), lambda i: (i, 0)),
        pl.BlockSpec((QT, 2 * C), lambda i: (i, 0)),
    ],
    out_shape=[
        jax.ShapeDtypeStruct((N, C), jnp.bfloat16),
        jax.ShapeDtypeStruct((N, 2 * C), jnp.bfloat16),
    ],
)


def _sc_gather_body(base_row, idx_hbm, tab_hbm, out_hbm, tab_v, idx_v, out_v,
                    in_sem, out_sem):
    wid = lax.axis_index("s") * 2 + lax.axis_index("c")
    row0 = wid * ROWS_W
    pltpu.sync_copy(tab_hbm, tab_v)

    def in_copy(hs, b):
        return pltpu.make_async_copy(
            idx_hbm.at[pl.ds(base_row + row0 + hs * SR, SR), :], idx_v.at[b],
            in_sem.at[b])

    def out_copy(hs, b):
        return pltpu.make_async_copy(
            out_v.at[b], out_hbm.at[:, pl.ds(row0 + hs * SR, SR), :],
            out_sem.at[b])

    in_copy(0, 0).start()
    for hs in range(NHS):
        b = hs % 2
        if hs + 1 < NHS:
            in_copy(hs + 1, 1 - b).start()
        in_copy(hs, b).wait()
        if hs >= 2:
            out_copy(hs - 2, b).wait()
        for r in range(SR):

            def vec_body(i, carry, _b=b, _r=r):
                ids = idx_v[_b, _r, pl.ds(i * 16, 16)]
                for j in range(NP):
                    g = plsc.load_gather(tab_v, [ids + (j * TP)])
                    out_v[_b, j, _r, pl.ds(i * 16, 16)] = g
                return carry

            lax.fori_loop(0, N // 16, vec_body, 0, unroll=4)
        out_copy(hs, b).start()
    out_copy(NHS - 2, 0).wait()
    out_copy(NHS - 1, 1).wait()


@functools.lru_cache(maxsize=None)
def _gather_call(split):
    return pl.kernel(
        functools.partial(_sc_gather_body, split * RSP),
        out_type=jax.ShapeDtypeStruct((NP, RSP, N), jnp.int32),
        mesh=plsc.VectorSubcoreMesh(core_axis_name="c", subcore_axis_name="s"),
        scratch_types=[
            pltpu.VMEM((NP * TP,), jnp.int32),
            pltpu.VMEM((2, SR, N), jnp.int32),
            pltpu.VMEM((2, NP, SR, N), jnp.int32),
            pltpu.SemaphoreType.DMA((2,)),
            pltpu.SemaphoreType.DMA((2,)),
        ],
        compiler_params=pltpu.CompilerParams(needs_layout_passes=False,
                                             use_tc_tiling_on_sc=True),
    )


def _attn_body(q_ref, kv_ref, bias_ref, wpe_ref, bpe_ref, bk_ref, bv_ref,
               wp_ref, bp_ref, out_ref):
    u0 = bias_ref[0]
    u1 = bias_ref[1]
    u2 = bias_ref[2]
    planes = (
        lax.bitcast_convert_type(lax.shift_left(u0, 16), jnp.float32),
        lax.bitcast_convert_type(lax.bitwise_and(u0, jnp.int32(-65536)),
                                 jnp.float32),
        lax.bitcast_convert_type(lax.shift_left(u1, 16), jnp.float32),
        lax.bitcast_convert_type(lax.bitwise_and(u1, jnp.int32(-65536)),
                                 jnp.float32),
        lax.bitcast_convert_type(lax.shift_left(u2, 16), jnp.float32),
    )
    outs = []
    for h in range(H):
        qh = q_ref[:, h * CH:(h + 1) * CH]
        kh = kv_ref[:, h * 2 * CH:h * 2 * CH + CH]
        vh = kv_ref[:, h * 2 * CH + CH:(h + 1) * 2 * CH]
        s = lax.dot_general(qh, kh, (((1,), (1,)), ((), ())),
                            preferred_element_type=jnp.float32) * SCALE
        b_h = planes[0] * wpe_ref[0, h] + planes[1] * wpe_ref[1, h]
        b_h += planes[2] * wpe_ref[2, h] + planes[3] * wpe_ref[3, h]
        b_h += planes[4] * wpe_ref[4, h] + bpe_ref[0, h]
        s = s + b_h
        bkh = bk_ref[:, h * CH:(h + 1) * CH]
        sbl = jnp.sum(qh.astype(jnp.float32) * bkh, axis=1,
                      keepdims=True) * SCALE
        m = jnp.maximum(jnp.max(s, axis=1, keepdims=True), sbl)
        pf = jnp.exp(s - m)
        pb = jnp.exp(sbl - m)
        den = jnp.sum(pf, axis=1, keepdims=True) + pb
        p = pf.astype(jnp.bfloat16)
        o = lax.dot_general(p, vh, (((1,), (0,)), ((), ())),
                            preferred_element_type=jnp.float32)
        o = (o + pb * bv_ref[:, h * CH:(h + 1) * CH]) / den
        outs.append(o.astype(jnp.bfloat16))
    oc = jnp.concatenate(outs, axis=1)
    out_ref[...] = lax.dot_general(oc, wp_ref[...], (((1,), (0,)), ((), ())),
                                   preferred_element_type=jnp.float32) + bp_ref[...]


@functools.lru_cache(maxsize=None)
def _attn_call(split):
    off = split * (RSP // QT)
    return pl.pallas_call(
        _attn_body,
        grid=(RSP // QT,),
        in_specs=[
            pl.BlockSpec((QT, C), lambda i, _o=off: (i + _o, 0)),
            pl.BlockSpec((N, 2 * C), lambda i: (0, 0)),
            pl.BlockSpec((NP, QT, N), lambda i: (0, i, 0)),
            pl.BlockSpec((5, H), lambda i: (0, 0)),
            pl.BlockSpec((1, H), lambda i: (0, 0)),
            pl.BlockSpec((1, C), lambda i: (0, 0)),
            pl.BlockSpec((1, C), lambda i: (0, 0)),
            pl.BlockSpec((C, C), lambda i: (0, 0)),
            pl.BlockSpec((1, C), lambda i: (0, 0)),
        ],
        out_specs=pl.BlockSpec((QT, C), lambda i: (i, 0)),
        out_shape=jax.ShapeDtypeStruct((RSP, C), jnp.float32),
    )


def kernel(feat, member_idx, cluster_mask, pe_idx, global_attn, Wq, bq, Wkv,
           bkv, blank_k, blank_v, Wpe, bpe, Wproj, bproj, pre_table):
    del member_idx, cluster_mask, global_attn
    feat2 = feat.reshape(N, C)
    q, kv = _proj_call(feat2, Wq, bq.reshape(1, C), Wkv,
                       bkv.reshape(1, 2 * C))
    pre_bf = jnp.pad(pre_table, ((0, TP - T), (0, 1))).astype(jnp.bfloat16)
    u16 = lax.bitcast_convert_type(pre_bf, jnp.uint16).astype(jnp.uint32)
    tab = (u16[:, 0::2] | (u16[:, 1::2] << 16)).astype(jnp.int32).T
    idx2 = pe_idx.reshape(N, N)
    tab_flat = tab.reshape(NP * TP)
    packs = [_gather_call(s)(idx2, tab_flat) for s in range(NSPLIT)]
    wproj_bf = Wproj.astype(jnp.bfloat16)
    outs = [
        _attn_call(s)(q, kv, packs[s], Wpe, bpe.reshape(1, H),
                      blank_k.reshape(1, C), blank_v.reshape(1, C), wproj_bf,
                      bproj.reshape(1, C)) for s in range(NSPLIT)
    ]
    out = jnp.concatenate(outs, axis=0)
    return out.reshape(1, N, C)
